# baseline (speedup 1.0000x reference)
import functools
import jax
import jax.numpy as jnp
from jax import lax
from jax.experimental import pallas as pl
from jax.experimental.pallas import tpu as pltpu
from jax.experimental.pallas import tpu_sc as plsc

B = 4096
HID = 128
NC = 5
K = 8
ALPHA = 0.5
BLK = 128
NBLK = B // BLK
BIGF = 1e30

NWORK = 32
ROWS_W = B // NWORK
CH = 32
NCHUNK = ROWS_W // CH
D = 2 * HID


def _dotT(a, b):
    return jax.lax.dot_general(a, b, (((1,), (1,)), ((), ())),
                               preferred_element_type=jnp.float32)


def _dot(a, b):
    return jax.lax.dot_general(a, b, (((1,), (0,)), ((), ())),
                               preferred_element_type=jnp.float32)


def _prep_body(x_ref, lvw_ref, lvb_ref, ltw_ref, ltb_ref,
               pvw_ref, pvb_ref, ptw_ref, ptb_ref, base_ref, baseh_ref):
    xv = x_ref[:, 0:512]
    xt = x_ref[:, 512:768]

    def ln(v, w, b):
        mu = jnp.mean(v, axis=1, keepdims=True)
        var = jnp.mean((v - mu) ** 2, axis=1, keepdims=True)
        return (v - mu) / jnp.sqrt(var + 1e-5) * w + b

    nv = ln(xv, lvw_ref[...], lvb_ref[...])
    nt = ln(xt, ltw_ref[...], ltb_ref[...])
    fv = _dotT(nv, pvw_ref[...]) + pvb_ref[...]
    ft = _dotT(nt, ptw_ref[...]) + ptb_ref[...]
    base = jnp.concatenate([fv, ft], axis=1)
    base_ref[...] = base
    baseh_ref[...] = base.astype(jnp.bfloat16)


def _core_body(xc_ref, yc_ref, uc_ref, xr_ref, yr_ref, ur_ref,
               basef_ref, baseb_ref,
               cphw_ref, cphb_ref, c1w_ref, c1b_ref, c2w_ref, c2b_ref,
               g1a_ref, g1b_ref, g1c_ref, g1bias_ref, g2w_ref, g2b_ref,
               idx_ref, wts_ref, gated_ref, ent_ref):
    pid = pl.program_id(0)
    dx = xc_ref[...] - xr_ref[...]
    dy = yc_ref[...] - yr_ref[...]
    d2 = dx * dx + dy * dy

    jota = jax.lax.broadcasted_iota(jnp.int32, (BLK, B), 1)
    row_id = jax.lax.broadcasted_iota(jnp.int32, (BLK, B), 0) + pid * BLK
    notself = jota != row_id
    eq = uc_ref[...] == ur_ref[...]
    same = jnp.logical_and(eq, notself)
    has_n = jnp.sum(same.astype(jnp.float32), axis=1, keepdims=True) > 0.0
    valid = jnp.logical_and(notself, jnp.logical_or(~has_n, same))
    d2m = jnp.where(valid, d2, BIGF)
    fiota = jota.astype(jnp.float32)

    v1 = None
    z = jnp.zeros((BLK, 1), jnp.float32)
    wn = jnp.zeros((BLK, B), jnp.float32)
    vals = []
    idxs = []
    for k in range(K):
        m = jnp.min(d2m, axis=1, keepdims=True)
        cand = jnp.where(d2m == m, fiota, BIGF)
        j = jnp.min(cand, axis=1, keepdims=True)
        onehot = fiota == j
        vk = -jnp.sqrt(jnp.maximum(m, 1e-12))
        if k == 0:
            v1 = vk
            ek = jnp.ones((BLK, 1), jnp.float32)
        else:
            ek = jnp.exp(vk - v1)
        z = z + ek
        wn = wn + jnp.where(onehot, ek, 0.0)
        vals.append(ek)
        idxs.append(j)
        if k < K - 1:
            d2m = jnp.where(onehot, BIGF, d2m)

    kiota = jax.lax.broadcasted_iota(jnp.int32, (BLK, K), 1)
    wblk = jnp.zeros((BLK, K), jnp.float32)
    iblk = jnp.zeros((BLK, K), jnp.float32)
    for k in range(K):
        wblk = jnp.where(kiota == k, vals[k] / z, wblk)
        iblk = jnp.where(kiota == k, idxs[k], iblk)
    idx_ref[...] = iblk.astype(jnp.int32)
    wts_ref[...] = wblk

    neigh = _dot(wn.astype(jnp.bfloat16), basef_ref[...]) / z
    tok = _dotT(jnp.maximum(_dotT(neigh, c1w_ref[...]) + c1b_ref[...], 0.0),
                c2w_ref[...]) + c2b_ref[...]

    base = baseb_ref[...]
    cpl = _dotT(base, cphw_ref[...]) + cphb_ref[...]
    cpl = cpl - jnp.max(cpl, axis=1, keepdims=True)
    cpe = jnp.exp(cpl)
    cp = cpe / jnp.sum(cpe, axis=1, keepdims=True)

    gh = (_dotT(base, g1a_ref[...]) + _dotT(cp, g1b_ref[...])
          + _dotT(tok, g1c_ref[...]) + g1bias_ref[...])
    gh = jnp.maximum(gh, 0.0)
    gl = _dotT(gh, g2w_ref[...]) + g2b_ref[...]
    gl = gl - jnp.max(gl, axis=1, keepdims=True)
    ge = jnp.exp(gl)
    gp = ge / jnp.sum(ge, axis=1, keepdims=True)

    ent_ref[...] = -jnp.sum(gp * jnp.log(gp + 1e-8), axis=1, keepdims=True)

    cols = jax.lax.broadcasted_iota(jnp.int32, (BLK, 2 * HID), 1)
    factor = jnp.where(cols < HID, gp[:, 0:1], gp[:, 1:2])
    gated_ref[...] = base * factor


def _sc_gather_body(feat_hbm, idxf_hbm, wtsf_hbm, out_hbm,
                    idx_v, wts_v, rows_v, out_v, sem):
    wid = lax.axis_index("s") * 2 + lax.axis_index("c")
    row0 = wid * ROWS_W
    pltpu.sync_copy(wtsf_hbm.at[pl.ds(row0 * K, ROWS_W * K)], wts_v)

    def chunk(c, _):
        pltpu.sync_copy(idxf_hbm.at[pl.ds(row0 * K + c * CH * K, CH * K)],
                        idx_v)
        pltpu.async_copy(feat_hbm.at[idx_v], rows_v, sem).wait()

        def rowpair(rp, _):
            wv = wts_v[pl.ds((c * CH + 2 * rp) * K, 2 * K)]
            for r01 in range(2):
                for k in range(K):
                    w = wv[r01 * K + k]
                    for h in range(D // 16):
                        seg = rows_v[(2 * rp + r01) * K + k,
                                     pl.ds(h * 16, 16)]
                        if k == 0:
                            out_v[2 * rp + r01, pl.ds(h * 16, 16)] = w * seg
                        else:
                            out_v[2 * rp + r01, pl.ds(h * 16, 16)] += w * seg
            return 0

        lax.fori_loop(0, CH // 2, rowpair, 0)
        pltpu.sync_copy(out_v, out_hbm.at[pl.ds(row0 + c * CH, CH)])
        return 0

    lax.fori_loop(0, NCHUNK, chunk, 0)


def _sc_gather(feat, idx, wts):
    mesh = plsc.VectorSubcoreMesh(core_axis_name="c", subcore_axis_name="s")
    fn = functools.partial(
        pl.kernel,
        mesh=mesh,
        out_type=jax.ShapeDtypeStruct((B, D), jnp.float32),
        scratch_types=[
            pltpu.VMEM((CH * K,), jnp.int32),
            pltpu.VMEM((ROWS_W * K,), jnp.float32),
            pltpu.VMEM((CH * K, D), jnp.float32),
            pltpu.VMEM((CH, D), jnp.float32),
            pltpu.SemaphoreType.DMA,
        ],
    )(_sc_gather_body)
    return fn(feat, idx.reshape(B * K), wts.reshape(B * K))


def _head_body(upd_ref, gatedb_ref,
               gu1w_ref, gu1b_ref, gu2w_ref, gu2b_ref,
               cls1w_ref, cls1b_ref, bng_ref, bnb_ref,
               cls2w_ref, cls2b_ref, out_ref):
    upd = upd_ref[...]
    upd = _dotT(jnp.maximum(_dotT(upd, gu1w_ref[...]) + gu1b_ref[...], 0.0),
                gu2w_ref[...]) + gu2b_ref[...]
    fused = gatedb_ref[...] + ALPHA * upd
    h = _dotT(fused, cls1w_ref[...]) + cls1b_ref[...]
    h = (h / jnp.sqrt(1.0 + 1e-5)) * bng_ref[...] + bnb_ref[...]
    h = jnp.maximum(h, 0.0)
    out_ref[...] = _dotT(h, cls2w_ref[...]) + cls2b_ref[...]


def _full(shape):
    return pl.BlockSpec(shape, lambda i: (0, 0))


def _rows(w):
    return pl.BlockSpec((BLK, w), lambda i: (i, 0))


@jax.jit
def kernel(x, ln_v_w, ln_v_b, ln_t_w, ln_t_b, proj_v_w, proj_v_b, proj_t_w,
           proj_t_b, cph_w, cph_b, ctx1_w, ctx1_b, ctx2_w, ctx2_b, g1_w,
           g1_b, g2_w, g2_b, gu1_w, gu1_b, gu2_w, gu2_b, cls1_w, cls1_b,
           bn_g, bn_b, cls2_w, cls2_b):
    xc = x[:, 768:769]
    yc = x[:, 769:770]
    uc = x[:, 772:773]
    xr = xc.reshape(1, B)
    yr = yc.reshape(1, B)
    ur = uc.reshape(1, B)

    r1 = lambda v: v.reshape(1, -1)

    base, baseh = pl.pallas_call(
        _prep_body,
        grid=(NBLK,),
        in_specs=[_rows(773)] + [_full((1, 512))] * 2
                 + [_full((1, 256))] * 2
                 + [_full((HID, 512)), _full((1, HID)),
                    _full((HID, 256)), _full((1, HID))],
        out_specs=[_rows(2 * HID), _rows(2 * HID)],
        out_shape=[jax.ShapeDtypeStruct((B, 2 * HID), jnp.float32),
                   jax.ShapeDtypeStruct((B, 2 * HID), jnp.bfloat16)],
    )(x, r1(ln_v_w), r1(ln_v_b), r1(ln_t_w), r1(ln_t_b),
      proj_v_w, r1(proj_v_b), proj_t_w, r1(proj_t_b))

    g1a = g1_w[:, 0:2 * HID]
    g1b = g1_w[:, 2 * HID:2 * HID + NC]
    g1c = g1_w[:, 2 * HID + NC:]

    idx, wts, gated, ent = pl.pallas_call(
        _core_body,
        grid=(NBLK,),
        in_specs=[_rows(1)] * 3 + [_full((1, B))] * 3
                 + [_full((B, 2 * HID)), _rows(2 * HID),
                    _full((NC, 2 * HID)), _full((1, NC)),
                    _full((HID, 2 * HID)), _full((1, HID)),
                    _full((HID, HID)), _full((1, HID)),
                    _full((128, 2 * HID)), _full((128, NC)),
                    _full((128, HID)), _full((1, 128)),
                    _full((2, 128)), _full((1, 2))],
        out_specs=[_rows(K), _rows(K), _rows(2 * HID), _rows(1)],
        out_shape=[jax.ShapeDtypeStruct((B, K), jnp.int32),
                   jax.ShapeDtypeStruct((B, K), jnp.float32),
                   jax.ShapeDtypeStruct((B, 2 * HID), jnp.float32),
                   jax.ShapeDtypeStruct((B, 1), jnp.float32)],
    )(xc, yc, uc, xr, yr, ur, baseh, base, cph_w, r1(cph_b),
      ctx1_w, r1(ctx1_b), ctx2_w, r1(ctx2_b),
      g1a, g1b, g1c, r1(g1_b), g2_w, r1(g2_b))

    upd_raw = _sc_gather(gated, idx, wts)

    logits = pl.pallas_call(
        _head_body,
        grid=(NBLK,),
        in_specs=[_rows(2 * HID), _rows(2 * HID),
                  _full((2 * HID, 2 * HID)), _full((1, 2 * HID)),
                  _full((2 * HID, 2 * HID)), _full((1, 2 * HID)),
                  _full((HID, 2 * HID)), _full((1, HID)),
                  _full((1, HID)), _full((1, HID)),
                  _full((NC, HID)), _full((1, NC))],
        out_specs=_rows(NC),
        out_shape=jax.ShapeDtypeStruct((B, NC), jnp.float32),
    )(upd_raw, gated, gu1_w, r1(gu1_b), gu2_w, r1(gu2_b),
      cls1_w, r1(cls1_b), r1(bn_g), r1(bn_b), cls2_w, r1(cls2_b))

    ent_loss = jnp.mean(ent) * 0.01
    return logits, ent_loss


# speedup vs baseline: 1.1626x; 1.1626x over previous
import functools
import jax
import jax.numpy as jnp
from jax import lax
from jax.experimental import pallas as pl
from jax.experimental.pallas import tpu as pltpu
from jax.experimental.pallas import tpu_sc as plsc

B = 4096
HID = 128
NC = 5
K = 8
ALPHA = 0.5
BLK = 128
NBLK = B // BLK
BIGF = 1e30

NWORK = 32
ROWS_W = B // NWORK
CH = 32
NCHUNK = ROWS_W // CH
D = 2 * HID


def _dotT(a, b):
    return jax.lax.dot_general(a, b, (((1,), (1,)), ((), ())),
                               preferred_element_type=jnp.float32)


def _dot(a, b):
    return jax.lax.dot_general(a, b, (((1,), (0,)), ((), ())),
                               preferred_element_type=jnp.float32)


def _prep_body(x_ref, lvw_ref, lvb_ref, ltw_ref, ltb_ref,
               pvw_ref, pvb_ref, ptw_ref, ptb_ref, base_ref, baseh_ref):
    xv = x_ref[:, 0:512]
    xt = x_ref[:, 512:768]

    def ln(v, w, b):
        mu = jnp.mean(v, axis=1, keepdims=True)
        var = jnp.mean((v - mu) ** 2, axis=1, keepdims=True)
        return (v - mu) / jnp.sqrt(var + 1e-5) * w + b

    nv = ln(xv, lvw_ref[...], lvb_ref[...])
    nt = ln(xt, ltw_ref[...], ltb_ref[...])
    fv = _dotT(nv, pvw_ref[...]) + pvb_ref[...]
    ft = _dotT(nt, ptw_ref[...]) + ptb_ref[...]
    base = jnp.concatenate([fv, ft], axis=1)
    base_ref[...] = base
    baseh_ref[...] = base.astype(jnp.bfloat16)


def _core_body(xc_ref, yc_ref, uc_ref, xr_ref, yr_ref, ur_ref,
               basef_ref, baseb_ref,
               cphw_ref, cphb_ref, c1w_ref, c1b_ref, c2w_ref, c2b_ref,
               g1a_ref, g1b_ref, g1c_ref, g1bias_ref, g2w_ref, g2b_ref,
               idx_ref, wts_ref, gated_ref, ent_ref):
    pid = pl.program_id(0)
    dx = xc_ref[...] - xr_ref[...]
    dy = yc_ref[...] - yr_ref[...]
    d2 = dx * dx + dy * dy

    jota = jax.lax.broadcasted_iota(jnp.int32, (BLK, B), 1)
    row_id = jax.lax.broadcasted_iota(jnp.int32, (BLK, B), 0) + pid * BLK
    notself = jota != row_id
    eq = uc_ref[...] == ur_ref[...]
    same = jnp.logical_and(eq, notself)
    has_n = jnp.sum(same.astype(jnp.float32), axis=1, keepdims=True) > 0.0
    valid = jnp.logical_and(notself, jnp.logical_or(~has_n, same))
    d2m = jnp.where(valid, d2, BIGF)
    fiota = jota.astype(jnp.float32)

    v1 = None
    z = jnp.zeros((BLK, 1), jnp.float32)
    wn = jnp.zeros((BLK, B), jnp.float32)
    vals = []
    idxs = []
    for k in range(K):
        m = jnp.min(d2m, axis=1, keepdims=True)
        cand = jnp.where(d2m == m, fiota, BIGF)
        j = jnp.min(cand, axis=1, keepdims=True)
        onehot = fiota == j
        vk = -jnp.sqrt(jnp.maximum(m, 1e-12))
        if k == 0:
            v1 = vk
            ek = jnp.ones((BLK, 1), jnp.float32)
        else:
            ek = jnp.exp(vk - v1)
        z = z + ek
        wn = wn + jnp.where(onehot, ek, 0.0)
        vals.append(ek)
        idxs.append(j)
        if k < K - 1:
            d2m = jnp.where(onehot, BIGF, d2m)

    kiota = jax.lax.broadcasted_iota(jnp.int32, (BLK, K), 1)
    wblk = jnp.zeros((BLK, K), jnp.float32)
    iblk = jnp.zeros((BLK, K), jnp.float32)
    for k in range(K):
        wblk = jnp.where(kiota == k, vals[k] / z, wblk)
        iblk = jnp.where(kiota == k, idxs[k], iblk)
    idx_ref[...] = iblk.astype(jnp.int32)
    wts_ref[...] = wblk

    neigh = _dot(wn.astype(jnp.bfloat16), basef_ref[...]) / z
    tok = _dotT(jnp.maximum(_dotT(neigh, c1w_ref[...]) + c1b_ref[...], 0.0),
                c2w_ref[...]) + c2b_ref[...]

    base = baseb_ref[...]
    cpl = _dotT(base, cphw_ref[...]) + cphb_ref[...]
    cpl = cpl - jnp.max(cpl, axis=1, keepdims=True)
    cpe = jnp.exp(cpl)
    cp = cpe / jnp.sum(cpe, axis=1, keepdims=True)

    gh = (_dotT(base, g1a_ref[...]) + _dotT(cp, g1b_ref[...])
          + _dotT(tok, g1c_ref[...]) + g1bias_ref[...])
    gh = jnp.maximum(gh, 0.0)
    gl = _dotT(gh, g2w_ref[...]) + g2b_ref[...]
    gl = gl - jnp.max(gl, axis=1, keepdims=True)
    ge = jnp.exp(gl)
    gp = ge / jnp.sum(ge, axis=1, keepdims=True)

    ent_ref[...] = -jnp.sum(gp * jnp.log(gp + 1e-8), axis=1, keepdims=True)

    cols = jax.lax.broadcasted_iota(jnp.int32, (BLK, 2 * HID), 1)
    factor = jnp.where(cols < HID, gp[:, 0:1], gp[:, 1:2])
    gated_ref[...] = base * factor


def _sc_gather_body(feat_hbm, idxf_hbm, wtsf_hbm, out_hbm,
                    idx_v, wts_v, rows_v, out_v, sem):
    wid = lax.axis_index("s") * 2 + lax.axis_index("c")
    row0 = wid * ROWS_W
    pltpu.sync_copy(wtsf_hbm.at[pl.ds(row0 * K, ROWS_W * K)], wts_v)

    def chunk(c, _):
        pltpu.sync_copy(idxf_hbm.at[pl.ds(row0 * K + c * CH * K, CH * K)],
                        idx_v)
        pltpu.async_copy(feat_hbm.at[idx_v], rows_v, sem).wait()

        def rowpair(rp, _):
            wv = wts_v[pl.ds((c * CH + 2 * rp) * K, 2 * K)]
            for r01 in range(2):
                ws = [wv[r01 * K + k] for k in range(K)]
                r = (2 * rp + r01) * K
                for h in range(D // 16):
                    acc = ws[0] * rows_v[r, pl.ds(h * 16, 16)]
                    for k in range(1, K):
                        acc = acc + ws[k] * rows_v[r + k, pl.ds(h * 16, 16)]
                    out_v[2 * rp + r01, pl.ds(h * 16, 16)] = acc
            return 0

        lax.fori_loop(0, CH // 2, rowpair, 0)
        pltpu.sync_copy(out_v, out_hbm.at[pl.ds(row0 + c * CH, CH)])
        return 0

    lax.fori_loop(0, NCHUNK, chunk, 0)


def _sc_gather(feat, idx, wts):
    mesh = plsc.VectorSubcoreMesh(core_axis_name="c", subcore_axis_name="s")
    fn = functools.partial(
        pl.kernel,
        mesh=mesh,
        out_type=jax.ShapeDtypeStruct((B, D), jnp.float32),
        scratch_types=[
            pltpu.VMEM((CH * K,), jnp.int32),
            pltpu.VMEM((ROWS_W * K,), jnp.float32),
            pltpu.VMEM((CH * K, D), jnp.float32),
            pltpu.VMEM((CH, D), jnp.float32),
            pltpu.SemaphoreType.DMA,
        ],
    )(_sc_gather_body)
    return fn(feat, idx.reshape(B * K), wts.reshape(B * K))


def _head_body(upd_ref, gatedb_ref,
               gu1w_ref, gu1b_ref, gu2w_ref, gu2b_ref,
               cls1w_ref, cls1b_ref, bng_ref, bnb_ref,
               cls2w_ref, cls2b_ref, out_ref):
    upd = upd_ref[...]
    upd = _dotT(jnp.maximum(_dotT(upd, gu1w_ref[...]) + gu1b_ref[...], 0.0),
                gu2w_ref[...]) + gu2b_ref[...]
    fused = gatedb_ref[...] + ALPHA * upd
    h = _dotT(fused, cls1w_ref[...]) + cls1b_ref[...]
    h = (h / jnp.sqrt(1.0 + 1e-5)) * bng_ref[...] + bnb_ref[...]
    h = jnp.maximum(h, 0.0)
    out_ref[...] = _dotT(h, cls2w_ref[...]) + cls2b_ref[...]


def _full(shape):
    return pl.BlockSpec(shape, lambda i: (0, 0))


def _rows(w):
    return pl.BlockSpec((BLK, w), lambda i: (i, 0))


@jax.jit
def kernel(x, ln_v_w, ln_v_b, ln_t_w, ln_t_b, proj_v_w, proj_v_b, proj_t_w,
           proj_t_b, cph_w, cph_b, ctx1_w, ctx1_b, ctx2_w, ctx2_b, g1_w,
           g1_b, g2_w, g2_b, gu1_w, gu1_b, gu2_w, gu2_b, cls1_w, cls1_b,
           bn_g, bn_b, cls2_w, cls2_b):
    xc = x[:, 768:769]
    yc = x[:, 769:770]
    uc = x[:, 772:773]
    xr = xc.reshape(1, B)
    yr = yc.reshape(1, B)
    ur = uc.reshape(1, B)

    r1 = lambda v: v.reshape(1, -1)

    base, baseh = pl.pallas_call(
        _prep_body,
        grid=(NBLK,),
        in_specs=[_rows(773)] + [_full((1, 512))] * 2
                 + [_full((1, 256))] * 2
                 + [_full((HID, 512)), _full((1, HID)),
                    _full((HID, 256)), _full((1, HID))],
        out_specs=[_rows(2 * HID), _rows(2 * HID)],
        out_shape=[jax.ShapeDtypeStruct((B, 2 * HID), jnp.float32),
                   jax.ShapeDtypeStruct((B, 2 * HID), jnp.bfloat16)],
    )(x, r1(ln_v_w), r1(ln_v_b), r1(ln_t_w), r1(ln_t_b),
      proj_v_w, r1(proj_v_b), proj_t_w, r1(proj_t_b))

    g1a = g1_w[:, 0:2 * HID]
    g1b = g1_w[:, 2 * HID:2 * HID + NC]
    g1c = g1_w[:, 2 * HID + NC:]

    idx, wts, gated, ent = pl.pallas_call(
        _core_body,
        grid=(NBLK,),
        in_specs=[_rows(1)] * 3 + [_full((1, B))] * 3
                 + [_full((B, 2 * HID)), _rows(2 * HID),
                    _full((NC, 2 * HID)), _full((1, NC)),
                    _full((HID, 2 * HID)), _full((1, HID)),
                    _full((HID, HID)), _full((1, HID)),
                    _full((128, 2 * HID)), _full((128, NC)),
                    _full((128, HID)), _full((1, 128)),
                    _full((2, 128)), _full((1, 2))],
        out_specs=[_rows(K), _rows(K), _rows(2 * HID), _rows(1)],
        out_shape=[jax.ShapeDtypeStruct((B, K), jnp.int32),
                   jax.ShapeDtypeStruct((B, K), jnp.float32),
                   jax.ShapeDtypeStruct((B, 2 * HID), jnp.float32),
                   jax.ShapeDtypeStruct((B, 1), jnp.float32)],
    )(xc, yc, uc, xr, yr, ur, baseh, base, cph_w, r1(cph_b),
      ctx1_w, r1(ctx1_b), ctx2_w, r1(ctx2_b),
      g1a, g1b, g1c, r1(g1_b), g2_w, r1(g2_b))

    upd_raw = _sc_gather(gated, idx, wts)

    logits = pl.pallas_call(
        _head_body,
        grid=(NBLK,),
        in_specs=[_rows(2 * HID), _rows(2 * HID),
                  _full((2 * HID, 2 * HID)), _full((1, 2 * HID)),
                  _full((2 * HID, 2 * HID)), _full((1, 2 * HID)),
                  _full((HID, 2 * HID)), _full((1, HID)),
                  _full((1, HID)), _full((1, HID)),
                  _full((NC, HID)), _full((1, NC))],
        out_specs=_rows(NC),
        out_shape=jax.ShapeDtypeStruct((B, NC), jnp.float32),
    )(upd_raw, gated, gu1_w, r1(gu1_b), gu2_w, r1(gu2_b),
      cls1_w, r1(cls1_b), r1(bn_g), r1(bn_b), cls2_w, r1(cls2_b))

    ent_loss = jnp.mean(ent) * 0.01
    return logits, ent_loss


# speedup vs baseline: 1.1693x; 1.0058x over previous
import functools
import jax
import jax.numpy as jnp
from jax import lax
from jax.experimental import pallas as pl
from jax.experimental.pallas import tpu as pltpu
from jax.experimental.pallas import tpu_sc as plsc

B = 4096
HID = 128
NC = 5
K = 8
ALPHA = 0.5
BLK = 128
NBLK = B // BLK
BIGF = 1e30

NWORK = 32
ROWS_W = B // NWORK
CH = 32
NCHUNK = ROWS_W // CH
D = 2 * HID


def _dotT(a, b):
    return jax.lax.dot_general(a, b, (((1,), (1,)), ((), ())),
                               preferred_element_type=jnp.float32)


def _dot(a, b):
    return jax.lax.dot_general(a, b, (((1,), (0,)), ((), ())),
                               preferred_element_type=jnp.float32)


def _prep_body(x_ref, lvw_ref, lvb_ref, ltw_ref, ltb_ref,
               pvw_ref, pvb_ref, ptw_ref, ptb_ref, base_ref, baseh_ref):
    xv = x_ref[:, 0:512]
    xt = x_ref[:, 512:768]

    def ln(v, w, b):
        mu = jnp.mean(v, axis=1, keepdims=True)
        var = jnp.mean((v - mu) ** 2, axis=1, keepdims=True)
        return (v - mu) / jnp.sqrt(var + 1e-5) * w + b

    nv = ln(xv, lvw_ref[...], lvb_ref[...])
    nt = ln(xt, ltw_ref[...], ltb_ref[...])
    fv = _dotT(nv, pvw_ref[...]) + pvb_ref[...]
    ft = _dotT(nt, ptw_ref[...]) + ptb_ref[...]
    base = jnp.concatenate([fv, ft], axis=1)
    base_ref[...] = base
    baseh_ref[...] = base.astype(jnp.bfloat16)


def _core_body(xc_ref, yc_ref, uc_ref, xr_ref, yr_ref, ur_ref,
               basef_ref, baseb_ref,
               cphw_ref, cphb_ref, c1w_ref, c1b_ref, c2w_ref, c2b_ref,
               g1a_ref, g1b_ref, g1c_ref, g1bias_ref, g2w_ref, g2b_ref,
               idx_ref, wts_ref, gated_ref, ent_ref):
    pid = pl.program_id(0)
    dx = xc_ref[...] - xr_ref[...]
    dy = yc_ref[...] - yr_ref[...]
    d2 = dx * dx + dy * dy

    jota = jax.lax.broadcasted_iota(jnp.int32, (BLK, B), 1)
    row_id = jax.lax.broadcasted_iota(jnp.int32, (BLK, B), 0) + pid * BLK
    notself = jota != row_id
    eq = uc_ref[...] == ur_ref[...]
    same = jnp.logical_and(eq, notself)
    has_n = jnp.sum(same.astype(jnp.float32), axis=1, keepdims=True) > 0.0
    valid = jnp.logical_and(notself, jnp.logical_or(~has_n, same))
    d2m = jnp.where(valid, d2, BIGF)
    fiota = jota.astype(jnp.float32)

    v1 = None
    z = jnp.zeros((BLK, 1), jnp.float32)
    wn = jnp.zeros((BLK, B), jnp.float32)
    vals = []
    idxs = []
    for k in range(K):
        m = jnp.min(d2m, axis=1, keepdims=True)
        cand = jnp.where(d2m == m, fiota, BIGF)
        j = jnp.min(cand, axis=1, keepdims=True)
        onehot = fiota == j
        vk = -jnp.sqrt(jnp.maximum(m, 1e-12))
        if k == 0:
            v1 = vk
            ek = jnp.ones((BLK, 1), jnp.float32)
        else:
            ek = jnp.exp(vk - v1)
        z = z + ek
        wn = wn + jnp.where(onehot, ek, 0.0)
        vals.append(ek)
        idxs.append(j)
        if k < K - 1:
            d2m = jnp.where(onehot, BIGF, d2m)

    kiota = jax.lax.broadcasted_iota(jnp.int32, (BLK, K), 1)
    wblk = jnp.zeros((BLK, K), jnp.float32)
    iblk = jnp.zeros((BLK, K), jnp.float32)
    for k in range(K):
        wblk = jnp.where(kiota == k, vals[k] / z, wblk)
        iblk = jnp.where(kiota == k, idxs[k], iblk)
    idx_ref[...] = iblk.astype(jnp.int32)
    wts_ref[...] = wblk

    neigh = _dot(wn.astype(jnp.bfloat16), basef_ref[...]) / z
    tok = _dotT(jnp.maximum(_dotT(neigh, c1w_ref[...]) + c1b_ref[...], 0.0),
                c2w_ref[...]) + c2b_ref[...]

    base = baseb_ref[...]
    cpl = _dotT(base, cphw_ref[...]) + cphb_ref[...]
    cpl = cpl - jnp.max(cpl, axis=1, keepdims=True)
    cpe = jnp.exp(cpl)
    cp = cpe / jnp.sum(cpe, axis=1, keepdims=True)

    gh = (_dotT(base, g1a_ref[...]) + _dotT(cp, g1b_ref[...])
          + _dotT(tok, g1c_ref[...]) + g1bias_ref[...])
    gh = jnp.maximum(gh, 0.0)
    gl = _dotT(gh, g2w_ref[...]) + g2b_ref[...]
    gl = gl - jnp.max(gl, axis=1, keepdims=True)
    ge = jnp.exp(gl)
    gp = ge / jnp.sum(ge, axis=1, keepdims=True)

    ent_ref[...] = -jnp.sum(gp * jnp.log(gp + 1e-8), axis=1, keepdims=True)

    cols = jax.lax.broadcasted_iota(jnp.int32, (BLK, 2 * HID), 1)
    factor = jnp.where(cols < HID, gp[:, 0:1], gp[:, 1:2])
    gated_ref[...] = base * factor


def _sc_gather_body(feat_hbm, idxf_hbm, wtsf_hbm, out_hbm,
                    idx_v, wts_v, rows_v, out_v, sem):
    wid = lax.axis_index("s") * 2 + lax.axis_index("c")
    row0 = wid * ROWS_W
    pltpu.sync_copy(wtsf_hbm.at[pl.ds(row0 * K, ROWS_W * K)], wts_v)

    def chunk(c, _):
        pltpu.sync_copy(idxf_hbm.at[pl.ds(row0 * K + c * CH * K, CH * K)],
                        idx_v)
        pltpu.async_copy(feat_hbm.at[idx_v], rows_v, sem).wait()

        def rowpair(rp, _):
            wv = wts_v[pl.ds((c * CH + 2 * rp) * K, 2 * K)]
            for r01 in range(2):
                ws = [wv[r01 * K + k] for k in range(K)]
                r = (2 * rp + r01) * K
                for h in range(D // 16):
                    hs = pl.ds(h * 16, 16)
                    p = [ws[2 * q] * rows_v[r + 2 * q, hs]
                         + ws[2 * q + 1] * rows_v[r + 2 * q + 1, hs]
                         for q in range(K // 2)]
                    out_v[2 * rp + r01, hs] = (p[0] + p[1]) + (p[2] + p[3])
            return 0

        lax.fori_loop(0, CH // 2, rowpair, 0)
        pltpu.sync_copy(out_v, out_hbm.at[pl.ds(row0 + c * CH, CH)])
        return 0

    lax.fori_loop(0, NCHUNK, chunk, 0)


def _sc_gather(feat, idx, wts):
    mesh = plsc.VectorSubcoreMesh(core_axis_name="c", subcore_axis_name="s")
    fn = functools.partial(
        pl.kernel,
        mesh=mesh,
        out_type=jax.ShapeDtypeStruct((B, D), jnp.float32),
        scratch_types=[
            pltpu.VMEM((CH * K,), jnp.int32),
            pltpu.VMEM((ROWS_W * K,), jnp.float32),
            pltpu.VMEM((CH * K, D), jnp.float32),
            pltpu.VMEM((CH, D), jnp.float32),
            pltpu.SemaphoreType.DMA,
        ],
    )(_sc_gather_body)
    return fn(feat, idx.reshape(B * K), wts.reshape(B * K))


def _head_body(upd_ref, gatedb_ref,
               gu1w_ref, gu1b_ref, gu2w_ref, gu2b_ref,
               cls1w_ref, cls1b_ref, bng_ref, bnb_ref,
               cls2w_ref, cls2b_ref, out_ref):
    upd = upd_ref[...]
    upd = _dotT(jnp.maximum(_dotT(upd, gu1w_ref[...]) + gu1b_ref[...], 0.0),
                gu2w_ref[...]) + gu2b_ref[...]
    fused = gatedb_ref[...] + ALPHA * upd
    h = _dotT(fused, cls1w_ref[...]) + cls1b_ref[...]
    h = (h / jnp.sqrt(1.0 + 1e-5)) * bng_ref[...] + bnb_ref[...]
    h = jnp.maximum(h, 0.0)
    out_ref[...] = _dotT(h, cls2w_ref[...]) + cls2b_ref[...]


def _full(shape):
    return pl.BlockSpec(shape, lambda i: (0, 0))


def _rows(w):
    return pl.BlockSpec((BLK, w), lambda i: (i, 0))


@jax.jit
def kernel(x, ln_v_w, ln_v_b, ln_t_w, ln_t_b, proj_v_w, proj_v_b, proj_t_w,
           proj_t_b, cph_w, cph_b, ctx1_w, ctx1_b, ctx2_w, ctx2_b, g1_w,
           g1_b, g2_w, g2_b, gu1_w, gu1_b, gu2_w, gu2_b, cls1_w, cls1_b,
           bn_g, bn_b, cls2_w, cls2_b):
    xc = x[:, 768:769]
    yc = x[:, 769:770]
    uc = x[:, 772:773]
    xr = xc.reshape(1, B)
    yr = yc.reshape(1, B)
    ur = uc.reshape(1, B)

    r1 = lambda v: v.reshape(1, -1)

    base, baseh = pl.pallas_call(
        _prep_body,
        grid=(NBLK,),
        in_specs=[_rows(773)] + [_full((1, 512))] * 2
                 + [_full((1, 256))] * 2
                 + [_full((HID, 512)), _full((1, HID)),
                    _full((HID, 256)), _full((1, HID))],
        out_specs=[_rows(2 * HID), _rows(2 * HID)],
        out_shape=[jax.ShapeDtypeStruct((B, 2 * HID), jnp.float32),
                   jax.ShapeDtypeStruct((B, 2 * HID), jnp.bfloat16)],
    )(x, r1(ln_v_w), r1(ln_v_b), r1(ln_t_w), r1(ln_t_b),
      proj_v_w, r1(proj_v_b), proj_t_w, r1(proj_t_b))

    g1a = g1_w[:, 0:2 * HID]
    g1b = g1_w[:, 2 * HID:2 * HID + NC]
    g1c = g1_w[:, 2 * HID + NC:]

    idx, wts, gated, ent = pl.pallas_call(
        _core_body,
        grid=(NBLK,),
        in_specs=[_rows(1)] * 3 + [_full((1, B))] * 3
                 + [_full((B, 2 * HID)), _rows(2 * HID),
                    _full((NC, 2 * HID)), _full((1, NC)),
                    _full((HID, 2 * HID)), _full((1, HID)),
                    _full((HID, HID)), _full((1, HID)),
                    _full((128, 2 * HID)), _full((128, NC)),
                    _full((128, HID)), _full((1, 128)),
                    _full((2, 128)), _full((1, 2))],
        out_specs=[_rows(K), _rows(K), _rows(2 * HID), _rows(1)],
        out_shape=[jax.ShapeDtypeStruct((B, K), jnp.int32),
                   jax.ShapeDtypeStruct((B, K), jnp.float32),
                   jax.ShapeDtypeStruct((B, 2 * HID), jnp.float32),
                   jax.ShapeDtypeStruct((B, 1), jnp.float32)],
    )(xc, yc, uc, xr, yr, ur, baseh, base, cph_w, r1(cph_b),
      ctx1_w, r1(ctx1_b), ctx2_w, r1(ctx2_b),
      g1a, g1b, g1c, r1(g1_b), g2_w, r1(g2_b))

    upd_raw = _sc_gather(gated, idx, wts)

    logits = pl.pallas_call(
        _head_body,
        grid=(NBLK,),
        in_specs=[_rows(2 * HID), _rows(2 * HID),
                  _full((2 * HID, 2 * HID)), _full((1, 2 * HID)),
                  _full((2 * HID, 2 * HID)), _full((1, 2 * HID)),
                  _full((HID, 2 * HID)), _full((1, HID)),
                  _full((1, HID)), _full((1, HID)),
                  _full((NC, HID)), _full((1, NC))],
        out_specs=_rows(NC),
        out_shape=jax.ShapeDtypeStruct((B, NC), jnp.float32),
    )(upd_raw, gated, gu1_w, r1(gu1_b), gu2_w, r1(gu2_b),
      cls1_w, r1(cls1_b), r1(bn_g), r1(bn_b), cls2_w, r1(cls2_b))

    ent_loss = jnp.mean(ent) * 0.01
    return logits, ent_loss


# speedup vs baseline: 1.1958x; 1.0227x over previous
import functools
import jax
import jax.numpy as jnp
from jax import lax
from jax.experimental import pallas as pl
from jax.experimental.pallas import tpu as pltpu
from jax.experimental.pallas import tpu_sc as plsc

B = 4096
HID = 128
NC = 5
K = 8
ALPHA = 0.5
BLK = 128
NBLK = B // BLK
BIGF = 1e30

NWORK = 32
ROWS_W = B // NWORK
CH = 32
NCHUNK = ROWS_W // CH
D = 2 * HID


def _dotT(a, b):
    return jax.lax.dot_general(a, b, (((1,), (1,)), ((), ())),
                               preferred_element_type=jnp.float32)


def _dot(a, b):
    return jax.lax.dot_general(a, b, (((1,), (0,)), ((), ())),
                               preferred_element_type=jnp.float32)


def _prep_body(x_ref, lvw_ref, lvb_ref, ltw_ref, ltb_ref,
               pvw_ref, pvb_ref, ptw_ref, ptb_ref, base_ref, baseh_ref):
    xv = x_ref[:, 0:512]
    xt = x_ref[:, 512:768]

    def ln(v, w, b):
        mu = jnp.mean(v, axis=1, keepdims=True)
        var = jnp.mean((v - mu) ** 2, axis=1, keepdims=True)
        return (v - mu) / jnp.sqrt(var + 1e-5) * w + b

    nv = ln(xv, lvw_ref[...], lvb_ref[...])
    nt = ln(xt, ltw_ref[...], ltb_ref[...])
    fv = _dotT(nv, pvw_ref[...]) + pvb_ref[...]
    ft = _dotT(nt, ptw_ref[...]) + ptb_ref[...]
    base = jnp.concatenate([fv, ft], axis=1)
    base_ref[...] = base
    baseh_ref[...] = base.astype(jnp.bfloat16)


def _core_body(xc_ref, yc_ref, uc_ref, xr_ref, yr_ref, ur_ref,
               basef_ref, baseb_ref,
               cphw_ref, cphb_ref, c1w_ref, c1b_ref, c2w_ref, c2b_ref,
               g1a_ref, g1b_ref, g1c_ref, g1bias_ref, g2w_ref, g2b_ref,
               idx_ref, wts_ref, gated_ref, ent_ref):
    pid = pl.program_id(0)
    dx = xc_ref[...] - xr_ref[...]
    dy = yc_ref[...] - yr_ref[...]
    d2 = dx * dx + dy * dy

    jota = jax.lax.broadcasted_iota(jnp.int32, (BLK, B), 1)
    row_id = jax.lax.broadcasted_iota(jnp.int32, (BLK, B), 0) + pid * BLK
    notself = jota != row_id
    eq = uc_ref[...] == ur_ref[...]
    same = jnp.logical_and(eq, notself)
    has_n = jnp.sum(same.astype(jnp.float32), axis=1, keepdims=True) > 0.0
    valid = jnp.logical_and(notself, jnp.logical_or(~has_n, same))
    d2m = jnp.where(valid, d2, BIGF)
    fiota = jota.astype(jnp.float32)

    v1 = None
    z = jnp.zeros((BLK, 1), jnp.float32)
    wn = jnp.zeros((BLK, B), jnp.float32)
    vals = []
    idxs = []
    for k in range(K):
        m = jnp.min(d2m, axis=1, keepdims=True)
        cand = jnp.where(d2m == m, fiota, BIGF)
        j = jnp.min(cand, axis=1, keepdims=True)
        onehot = fiota == j
        vk = -jnp.sqrt(jnp.maximum(m, 1e-12))
        if k == 0:
            v1 = vk
            ek = jnp.ones((BLK, 1), jnp.float32)
        else:
            ek = jnp.exp(vk - v1)
        z = z + ek
        wn = wn + jnp.where(onehot, ek, 0.0)
        vals.append(ek)
        idxs.append(j)
        if k < K - 1:
            d2m = jnp.where(onehot, BIGF, d2m)

    kiota = jax.lax.broadcasted_iota(jnp.int32, (BLK, K), 1)
    wblk = jnp.zeros((BLK, K), jnp.float32)
    iblk = jnp.zeros((BLK, K), jnp.float32)
    for k in range(K):
        wblk = jnp.where(kiota == k, vals[k] / z, wblk)
        iblk = jnp.where(kiota == k, idxs[k], iblk)
    idx_ref[...] = iblk.astype(jnp.int32)
    wts_ref[...] = wblk

    neigh = _dot(wn.astype(jnp.bfloat16), basef_ref[...]) / z
    tok = _dotT(jnp.maximum(_dotT(neigh, c1w_ref[...]) + c1b_ref[...], 0.0),
                c2w_ref[...]) + c2b_ref[...]

    base = baseb_ref[...]
    cpl = _dotT(base, cphw_ref[...]) + cphb_ref[...]
    cpl = cpl - jnp.max(cpl, axis=1, keepdims=True)
    cpe = jnp.exp(cpl)
    cp = cpe / jnp.sum(cpe, axis=1, keepdims=True)

    gh = (_dotT(base, g1a_ref[...]) + _dotT(cp, g1b_ref[...])
          + _dotT(tok, g1c_ref[...]) + g1bias_ref[...])
    gh = jnp.maximum(gh, 0.0)
    gl = _dotT(gh, g2w_ref[...]) + g2b_ref[...]
    gl = gl - jnp.max(gl, axis=1, keepdims=True)
    ge = jnp.exp(gl)
    gp = ge / jnp.sum(ge, axis=1, keepdims=True)

    ent_ref[...] = -jnp.sum(gp * jnp.log(gp + 1e-8), axis=1, keepdims=True)

    cols = jax.lax.broadcasted_iota(jnp.int32, (BLK, 2 * HID), 1)
    factor = jnp.where(cols < HID, gp[:, 0:1], gp[:, 1:2])
    gated_ref[...] = base * factor


def _sc_gather_body(feat_hbm, idxf_hbm, wtsf_hbm, out_hbm,
                    idx_v, wts_v, rows_v, out_v, sem):
    wid = lax.axis_index("s") * 2 + lax.axis_index("c")
    row0 = wid * ROWS_W
    pltpu.sync_copy(wtsf_hbm.at[pl.ds(row0 * K, ROWS_W * K)], wts_v)

    def chunk(c, _):
        pltpu.sync_copy(idxf_hbm.at[pl.ds(row0 * K + c * CH * K, CH * K)],
                        idx_v)
        pltpu.async_copy(feat_hbm.at[idx_v], rows_v, sem).wait()

        @plsc.parallel_loop(0, CH // 2, unroll=2)
        def rowpair(rp):
            wv = wts_v[pl.ds((c * CH + 2 * rp) * K, 2 * K)]
            for r01 in range(2):
                ws = [wv[r01 * K + k] for k in range(K)]
                r = (2 * rp + r01) * K
                for h in range(D // 16):
                    hs = pl.ds(h * 16, 16)
                    p = [ws[2 * q] * rows_v[r + 2 * q, hs]
                         + ws[2 * q + 1] * rows_v[r + 2 * q + 1, hs]
                         for q in range(K // 2)]
                    out_v[2 * rp + r01, hs] = (p[0] + p[1]) + (p[2] + p[3])
        pltpu.sync_copy(out_v, out_hbm.at[pl.ds(row0 + c * CH, CH)])
        return 0

    lax.fori_loop(0, NCHUNK, chunk, 0)


def _sc_gather(feat, idx, wts):
    mesh = plsc.VectorSubcoreMesh(core_axis_name="c", subcore_axis_name="s")
    fn = functools.partial(
        pl.kernel,
        mesh=mesh,
        out_type=jax.ShapeDtypeStruct((B, D), jnp.float32),
        scratch_types=[
            pltpu.VMEM((CH * K,), jnp.int32),
            pltpu.VMEM((ROWS_W * K,), jnp.float32),
            pltpu.VMEM((CH * K, D), jnp.float32),
            pltpu.VMEM((CH, D), jnp.float32),
            pltpu.SemaphoreType.DMA,
        ],
    )(_sc_gather_body)
    return fn(feat, idx.reshape(B * K), wts.reshape(B * K))


def _head_body(upd_ref, gatedb_ref,
               gu1w_ref, gu1b_ref, gu2w_ref, gu2b_ref,
               cls1w_ref, cls1b_ref, bng_ref, bnb_ref,
               cls2w_ref, cls2b_ref, out_ref):
    upd = upd_ref[...]
    upd = _dotT(jnp.maximum(_dotT(upd, gu1w_ref[...]) + gu1b_ref[...], 0.0),
                gu2w_ref[...]) + gu2b_ref[...]
    fused = gatedb_ref[...] + ALPHA * upd
    h = _dotT(fused, cls1w_ref[...]) + cls1b_ref[...]
    h = (h / jnp.sqrt(1.0 + 1e-5)) * bng_ref[...] + bnb_ref[...]
    h = jnp.maximum(h, 0.0)
    out_ref[...] = _dotT(h, cls2w_ref[...]) + cls2b_ref[...]


def _full(shape):
    return pl.BlockSpec(shape, lambda i: (0, 0))


def _rows(w):
    return pl.BlockSpec((BLK, w), lambda i: (i, 0))


@jax.jit
def kernel(x, ln_v_w, ln_v_b, ln_t_w, ln_t_b, proj_v_w, proj_v_b, proj_t_w,
           proj_t_b, cph_w, cph_b, ctx1_w, ctx1_b, ctx2_w, ctx2_b, g1_w,
           g1_b, g2_w, g2_b, gu1_w, gu1_b, gu2_w, gu2_b, cls1_w, cls1_b,
           bn_g, bn_b, cls2_w, cls2_b):
    xc = x[:, 768:769]
    yc = x[:, 769:770]
    uc = x[:, 772:773]
    xr = xc.reshape(1, B)
    yr = yc.reshape(1, B)
    ur = uc.reshape(1, B)

    r1 = lambda v: v.reshape(1, -1)

    base, baseh = pl.pallas_call(
        _prep_body,
        grid=(NBLK,),
        in_specs=[_rows(773)] + [_full((1, 512))] * 2
                 + [_full((1, 256))] * 2
                 + [_full((HID, 512)), _full((1, HID)),
                    _full((HID, 256)), _full((1, HID))],
        out_specs=[_rows(2 * HID), _rows(2 * HID)],
        out_shape=[jax.ShapeDtypeStruct((B, 2 * HID), jnp.float32),
                   jax.ShapeDtypeStruct((B, 2 * HID), jnp.bfloat16)],
    )(x, r1(ln_v_w), r1(ln_v_b), r1(ln_t_w), r1(ln_t_b),
      proj_v_w, r1(proj_v_b), proj_t_w, r1(proj_t_b))

    g1a = g1_w[:, 0:2 * HID]
    g1b = g1_w[:, 2 * HID:2 * HID + NC]
    g1c = g1_w[:, 2 * HID + NC:]

    idx, wts, gated, ent = pl.pallas_call(
        _core_body,
        grid=(NBLK,),
        in_specs=[_rows(1)] * 3 + [_full((1, B))] * 3
                 + [_full((B, 2 * HID)), _rows(2 * HID),
                    _full((NC, 2 * HID)), _full((1, NC)),
                    _full((HID, 2 * HID)), _full((1, HID)),
                    _full((HID, HID)), _full((1, HID)),
                    _full((128, 2 * HID)), _full((128, NC)),
                    _full((128, HID)), _full((1, 128)),
                    _full((2, 128)), _full((1, 2))],
        out_specs=[_rows(K), _rows(K), _rows(2 * HID), _rows(1)],
        out_shape=[jax.ShapeDtypeStruct((B, K), jnp.int32),
                   jax.ShapeDtypeStruct((B, K), jnp.float32),
                   jax.ShapeDtypeStruct((B, 2 * HID), jnp.float32),
                   jax.ShapeDtypeStruct((B, 1), jnp.float32)],
    )(xc, yc, uc, xr, yr, ur, baseh, base, cph_w, r1(cph_b),
      ctx1_w, r1(ctx1_b), ctx2_w, r1(ctx2_b),
      g1a, g1b, g1c, r1(g1_b), g2_w, r1(g2_b))

    upd_raw = _sc_gather(gated, idx, wts)

    logits = pl.pallas_call(
        _head_body,
        grid=(NBLK,),
        in_specs=[_rows(2 * HID), _rows(2 * HID),
                  _full((2 * HID, 2 * HID)), _full((1, 2 * HID)),
                  _full((2 * HID, 2 * HID)), _full((1, 2 * HID)),
                  _full((HID, 2 * HID)), _full((1, HID)),
                  _full((1, HID)), _full((1, HID)),
                  _full((NC, HID)), _full((1, NC))],
        out_specs=_rows(NC),
        out_shape=jax.ShapeDtypeStruct((B, NC), jnp.float32),
    )(upd_raw, gated, gu1_w, r1(gu1_b), gu2_w, r1(gu2_b),
      cls1_w, r1(cls1_b), r1(bn_g), r1(bn_b), cls2_w, r1(cls2_b))

    ent_loss = jnp.mean(ent) * 0.01
    return logits, ent_loss


# speedup vs baseline: 1.2143x; 1.0154x over previous
import functools
import jax
import jax.numpy as jnp
from jax import lax
from jax.experimental import pallas as pl
from jax.experimental.pallas import tpu as pltpu
from jax.experimental.pallas import tpu_sc as plsc

B = 4096
HID = 128
NC = 5
K = 8
ALPHA = 0.5
BLK = 128
NBLK = B // BLK
BIGF = 1e30

NWORK = 32
ROWS_W = B // NWORK
CH = 16
NCHUNK = ROWS_W // CH
D = 2 * HID


def _dotT(a, b):
    return jax.lax.dot_general(a, b, (((1,), (1,)), ((), ())),
                               preferred_element_type=jnp.float32)


def _dot(a, b):
    return jax.lax.dot_general(a, b, (((1,), (0,)), ((), ())),
                               preferred_element_type=jnp.float32)


def _prep_body(x_ref, lvw_ref, lvb_ref, ltw_ref, ltb_ref,
               pvw_ref, pvb_ref, ptw_ref, ptb_ref, base_ref, baseh_ref):
    xv = x_ref[:, 0:512]
    xt = x_ref[:, 512:768]

    def ln(v, w, b):
        mu = jnp.mean(v, axis=1, keepdims=True)
        var = jnp.mean((v - mu) ** 2, axis=1, keepdims=True)
        return (v - mu) / jnp.sqrt(var + 1e-5) * w + b

    nv = ln(xv, lvw_ref[...], lvb_ref[...])
    nt = ln(xt, ltw_ref[...], ltb_ref[...])
    fv = _dotT(nv, pvw_ref[...]) + pvb_ref[...]
    ft = _dotT(nt, ptw_ref[...]) + ptb_ref[...]
    base = jnp.concatenate([fv, ft], axis=1)
    base_ref[...] = base
    baseh_ref[...] = base.astype(jnp.bfloat16)


def _core_body(xc_ref, yc_ref, uc_ref, xr_ref, yr_ref, ur_ref,
               basef_ref, baseb_ref,
               cphw_ref, cphb_ref, c1w_ref, c1b_ref, c2w_ref, c2b_ref,
               g1a_ref, g1b_ref, g1c_ref, g1bias_ref, g2w_ref, g2b_ref,
               idx_ref, wts_ref, gated_ref, ent_ref):
    pid = pl.program_id(0)
    dx = xc_ref[...] - xr_ref[...]
    dy = yc_ref[...] - yr_ref[...]
    d2 = dx * dx + dy * dy

    jota = jax.lax.broadcasted_iota(jnp.int32, (BLK, B), 1)
    row_id = jax.lax.broadcasted_iota(jnp.int32, (BLK, B), 0) + pid * BLK
    notself = jota != row_id
    eq = uc_ref[...] == ur_ref[...]
    same = jnp.logical_and(eq, notself)
    has_n = jnp.sum(same.astype(jnp.float32), axis=1, keepdims=True) > 0.0
    valid = jnp.logical_and(notself, jnp.logical_or(~has_n, same))
    d2m = jnp.where(valid, d2, BIGF)
    fiota = jota.astype(jnp.float32)

    v1 = None
    z = jnp.zeros((BLK, 1), jnp.float32)
    wn = jnp.zeros((BLK, B), jnp.float32)
    vals = []
    idxs = []
    for k in range(K):
        m = jnp.min(d2m, axis=1, keepdims=True)
        cand = jnp.where(d2m == m, fiota, BIGF)
        j = jnp.min(cand, axis=1, keepdims=True)
        onehot = fiota == j
        vk = -jnp.sqrt(jnp.maximum(m, 1e-12))
        if k == 0:
            v1 = vk
            ek = jnp.ones((BLK, 1), jnp.float32)
        else:
            ek = jnp.exp(vk - v1)
        z = z + ek
        wn = wn + jnp.where(onehot, ek, 0.0)
        vals.append(ek)
        idxs.append(j)
        if k < K - 1:
            d2m = jnp.where(onehot, BIGF, d2m)

    kiota = jax.lax.broadcasted_iota(jnp.int32, (BLK, K), 1)
    wblk = jnp.zeros((BLK, K), jnp.float32)
    iblk = jnp.zeros((BLK, K), jnp.float32)
    for k in range(K):
        wblk = jnp.where(kiota == k, vals[k] / z, wblk)
        iblk = jnp.where(kiota == k, idxs[k], iblk)
    idx_ref[...] = iblk.astype(jnp.int32)
    wts_ref[...] = wblk

    neigh = _dot(wn.astype(jnp.bfloat16), basef_ref[...]) / z
    tok = _dotT(jnp.maximum(_dotT(neigh, c1w_ref[...]) + c1b_ref[...], 0.0),
                c2w_ref[...]) + c2b_ref[...]

    base = baseb_ref[...]
    cpl = _dotT(base, cphw_ref[...]) + cphb_ref[...]
    cpl = cpl - jnp.max(cpl, axis=1, keepdims=True)
    cpe = jnp.exp(cpl)
    cp = cpe / jnp.sum(cpe, axis=1, keepdims=True)

    gh = (_dotT(base, g1a_ref[...]) + _dotT(cp, g1b_ref[...])
          + _dotT(tok, g1c_ref[...]) + g1bias_ref[...])
    gh = jnp.maximum(gh, 0.0)
    gl = _dotT(gh, g2w_ref[...]) + g2b_ref[...]
    gl = gl - jnp.max(gl, axis=1, keepdims=True)
    ge = jnp.exp(gl)
    gp = ge / jnp.sum(ge, axis=1, keepdims=True)

    ent_ref[...] = -jnp.sum(gp * jnp.log(gp + 1e-8), axis=1, keepdims=True)

    cols = jax.lax.broadcasted_iota(jnp.int32, (BLK, 2 * HID), 1)
    factor = jnp.where(cols < HID, gp[:, 0:1], gp[:, 1:2])
    gated_ref[...] = base * factor


def _sc_gather_body(feat_hbm, idxf_hbm, wtsf_hbm, out_hbm,
                    idx0_v, idx1_v, wts_v, rows0_v, rows1_v, out_v,
                    sem0, sem1):
    wid = lax.axis_index("s") * 2 + lax.axis_index("c")
    row0 = wid * ROWS_W
    pltpu.sync_copy(wtsf_hbm.at[pl.ds(row0 * K, ROWS_W * K)], wts_v)

    idxb = [idx0_v, idx1_v]
    rowsb = [rows0_v, rows1_v]
    semb = [sem0, sem1]

    def issue(c, b):
        pltpu.sync_copy(idxf_hbm.at[pl.ds(row0 * K + c * CH * K, CH * K)],
                        idxb[b])
        pltpu.async_copy(feat_hbm.at[idxb[b]], rowsb[b], semb[b])

    issue(0, 0)

    def outer(c2, _):
        for b in range(2):
            c = c2 * 2 + b

            @pl.when(c + 1 < NCHUNK)
            def _():
                issue(c + 1, 1 - b)

            rows_v = rowsb[b]
            pltpu.make_async_copy(feat_hbm.at[idxb[b]], rows_v,
                                  semb[b]).wait()

            @plsc.parallel_loop(0, CH // 2, unroll=2)
            def rowpair(rp):
                wv = wts_v[pl.ds((c * CH + 2 * rp) * K, 2 * K)]
                for r01 in range(2):
                    ws = [wv[r01 * K + k] for k in range(K)]
                    r = (2 * rp + r01) * K
                    for h in range(D // 16):
                        hs = pl.ds(h * 16, 16)
                        p = [ws[2 * q] * rows_v[r + 2 * q, hs]
                             + ws[2 * q + 1] * rows_v[r + 2 * q + 1, hs]
                             for q in range(K // 2)]
                        out_v[2 * rp + r01, hs] = ((p[0] + p[1])
                                                   + (p[2] + p[3]))
            pltpu.sync_copy(out_v, out_hbm.at[pl.ds(row0 + c * CH, CH)])
        return 0

    lax.fori_loop(0, NCHUNK // 2, outer, 0)


def _sc_gather(feat, idx, wts):
    mesh = plsc.VectorSubcoreMesh(core_axis_name="c", subcore_axis_name="s")
    fn = functools.partial(
        pl.kernel,
        mesh=mesh,
        out_type=jax.ShapeDtypeStruct((B, D), jnp.float32),
        scratch_types=[
            pltpu.VMEM((CH * K,), jnp.int32),
            pltpu.VMEM((CH * K,), jnp.int32),
            pltpu.VMEM((ROWS_W * K,), jnp.float32),
            pltpu.VMEM((CH * K, D), jnp.float32),
            pltpu.VMEM((CH * K, D), jnp.float32),
            pltpu.VMEM((CH, D), jnp.float32),
            pltpu.SemaphoreType.DMA,
            pltpu.SemaphoreType.DMA,
        ],
    )(_sc_gather_body)
    return fn(feat, idx.reshape(B * K), wts.reshape(B * K))


def _head_body(upd_ref, gatedb_ref,
               gu1w_ref, gu1b_ref, gu2w_ref, gu2b_ref,
               cls1w_ref, cls1b_ref, bng_ref, bnb_ref,
               cls2w_ref, cls2b_ref, out_ref):
    upd = upd_ref[...]
    upd = _dotT(jnp.maximum(_dotT(upd, gu1w_ref[...]) + gu1b_ref[...], 0.0),
                gu2w_ref[...]) + gu2b_ref[...]
    fused = gatedb_ref[...] + ALPHA * upd
    h = _dotT(fused, cls1w_ref[...]) + cls1b_ref[...]
    h = (h / jnp.sqrt(1.0 + 1e-5)) * bng_ref[...] + bnb_ref[...]
    h = jnp.maximum(h, 0.0)
    out_ref[...] = _dotT(h, cls2w_ref[...]) + cls2b_ref[...]


def _full(shape):
    return pl.BlockSpec(shape, lambda i: (0, 0))


def _rows(w):
    return pl.BlockSpec((BLK, w), lambda i: (i, 0))


@jax.jit
def kernel(x, ln_v_w, ln_v_b, ln_t_w, ln_t_b, proj_v_w, proj_v_b, proj_t_w,
           proj_t_b, cph_w, cph_b, ctx1_w, ctx1_b, ctx2_w, ctx2_b, g1_w,
           g1_b, g2_w, g2_b, gu1_w, gu1_b, gu2_w, gu2_b, cls1_w, cls1_b,
           bn_g, bn_b, cls2_w, cls2_b):
    xc = x[:, 768:769]
    yc = x[:, 769:770]
    uc = x[:, 772:773]
    xr = xc.reshape(1, B)
    yr = yc.reshape(1, B)
    ur = uc.reshape(1, B)

    r1 = lambda v: v.reshape(1, -1)

    base, baseh = pl.pallas_call(
        _prep_body,
        grid=(NBLK,),
        in_specs=[_rows(773)] + [_full((1, 512))] * 2
                 + [_full((1, 256))] * 2
                 + [_full((HID, 512)), _full((1, HID)),
                    _full((HID, 256)), _full((1, HID))],
        out_specs=[_rows(2 * HID), _rows(2 * HID)],
        out_shape=[jax.ShapeDtypeStruct((B, 2 * HID), jnp.float32),
                   jax.ShapeDtypeStruct((B, 2 * HID), jnp.bfloat16)],
    )(x, r1(ln_v_w), r1(ln_v_b), r1(ln_t_w), r1(ln_t_b),
      proj_v_w, r1(proj_v_b), proj_t_w, r1(proj_t_b))

    g1a = g1_w[:, 0:2 * HID]
    g1b = g1_w[:, 2 * HID:2 * HID + NC]
    g1c = g1_w[:, 2 * HID + NC:]

    idx, wts, gated, ent = pl.pallas_call(
        _core_body,
        grid=(NBLK,),
        in_specs=[_rows(1)] * 3 + [_full((1, B))] * 3
                 + [_full((B, 2 * HID)), _rows(2 * HID),
                    _full((NC, 2 * HID)), _full((1, NC)),
                    _full((HID, 2 * HID)), _full((1, HID)),
                    _full((HID, HID)), _full((1, HID)),
                    _full((128, 2 * HID)), _full((128, NC)),
                    _full((128, HID)), _full((1, 128)),
                    _full((2, 128)), _full((1, 2))],
        out_specs=[_rows(K), _rows(K), _rows(2 * HID), _rows(1)],
        out_shape=[jax.ShapeDtypeStruct((B, K), jnp.int32),
                   jax.ShapeDtypeStruct((B, K), jnp.float32),
                   jax.ShapeDtypeStruct((B, 2 * HID), jnp.float32),
                   jax.ShapeDtypeStruct((B, 1), jnp.float32)],
    )(xc, yc, uc, xr, yr, ur, baseh, base, cph_w, r1(cph_b),
      ctx1_w, r1(ctx1_b), ctx2_w, r1(ctx2_b),
      g1a, g1b, g1c, r1(g1_b), g2_w, r1(g2_b))

    upd_raw = _sc_gather(gated, idx, wts)

    logits = pl.pallas_call(
        _head_body,
        grid=(NBLK,),
        in_specs=[_rows(2 * HID), _rows(2 * HID),
                  _full((2 * HID, 2 * HID)), _full((1, 2 * HID)),
                  _full((2 * HID, 2 * HID)), _full((1, 2 * HID)),
                  _full((HID, 2 * HID)), _full((1, HID)),
                  _full((1, HID)), _full((1, HID)),
                  _full((NC, HID)), _full((1, NC))],
        out_specs=_rows(NC),
        out_shape=jax.ShapeDtypeStruct((B, NC), jnp.float32),
    )(upd_raw, gated, gu1_w, r1(gu1_b), gu2_w, r1(gu2_b),
      cls1_w, r1(cls1_b), r1(bn_g), r1(bn_b), cls2_w, r1(cls2_b))

    ent_loss = jnp.mean(ent) * 0.01
    return logits, ent_loss


# speedup vs baseline: 1.3598x; 1.1198x over previous
import functools
import jax
import jax.numpy as jnp
from jax import lax
from jax.experimental import pallas as pl
from jax.experimental.pallas import tpu as pltpu
from jax.experimental.pallas import tpu_sc as plsc

B = 4096
HID = 128
NC = 5
K = 8
ALPHA = 0.5
BLK = 256
NBLK = B // BLK
BIGF = 1e30

NWORK = 32
ROWS_W = B // NWORK
CH = 16
NCHUNK = ROWS_W // CH
D = 2 * HID


def _dotT(a, b):
    return jax.lax.dot_general(a, b, (((1,), (1,)), ((), ())),
                               preferred_element_type=jnp.float32)


def _dot(a, b):
    return jax.lax.dot_general(a, b, (((1,), (0,)), ((), ())),
                               preferred_element_type=jnp.float32)


def _prep_body(x_ref, lvw_ref, lvb_ref, ltw_ref, ltb_ref,
               pvw_ref, pvb_ref, ptw_ref, ptb_ref, base_ref, baseh_ref):
    xv = x_ref[:, 0:512]
    xt = x_ref[:, 512:768]

    def ln(v, w, b):
        mu = jnp.mean(v, axis=1, keepdims=True)
        var = jnp.mean((v - mu) ** 2, axis=1, keepdims=True)
        return (v - mu) / jnp.sqrt(var + 1e-5) * w + b

    nv = ln(xv, lvw_ref[...], lvb_ref[...])
    nt = ln(xt, ltw_ref[...], ltb_ref[...])
    fv = _dotT(nv, pvw_ref[...]) + pvb_ref[...]
    ft = _dotT(nt, ptw_ref[...]) + ptb_ref[...]
    base = jnp.concatenate([fv, ft], axis=1)
    base_ref[...] = base
    baseh_ref[...] = base.astype(jnp.bfloat16)


def _core_body(xc_ref, yc_ref, uc_ref, xr_ref, yr_ref, ur_ref,
               basef_ref, baseb_ref,
               cphw_ref, cphb_ref, c1w_ref, c1b_ref, c2w_ref, c2b_ref,
               g1a_ref, g1b_ref, g1c_ref, g1bias_ref, g2w_ref, g2b_ref,
               idx_ref, wts_ref, gated_ref, ent_ref):
    pid = pl.program_id(0)
    dx = xc_ref[...] - xr_ref[...]
    dy = yc_ref[...] - yr_ref[...]
    d2 = dx * dx + dy * dy

    jota = jax.lax.broadcasted_iota(jnp.int32, (BLK, B), 1)
    row_id = jax.lax.broadcasted_iota(jnp.int32, (BLK, B), 0) + pid * BLK
    notself = jota != row_id
    eq = uc_ref[...] == ur_ref[...]
    same = jnp.logical_and(eq, notself)
    has_n = jnp.sum(same.astype(jnp.float32), axis=1, keepdims=True) > 0.0
    valid = jnp.logical_and(notself, jnp.logical_or(~has_n, same))
    d2m = jnp.where(valid, d2, BIGF)
    fiota = jota.astype(jnp.float32)

    v1 = None
    z = jnp.zeros((BLK, 1), jnp.float32)
    wn = jnp.zeros((BLK, B), jnp.float32)
    vals = []
    idxs = []
    for k in range(K):
        m = jnp.min(d2m, axis=1, keepdims=True)
        cand = jnp.where(d2m == m, fiota, BIGF)
        j = jnp.min(cand, axis=1, keepdims=True)
        onehot = fiota == j
        vk = -jnp.sqrt(jnp.maximum(m, 1e-12))
        if k == 0:
            v1 = vk
            ek = jnp.ones((BLK, 1), jnp.float32)
        else:
            ek = jnp.exp(vk - v1)
        z = z + ek
        wn = wn + jnp.where(onehot, ek, 0.0)
        vals.append(ek)
        idxs.append(j)
        if k < K - 1:
            d2m = jnp.where(onehot, BIGF, d2m)

    kiota = jax.lax.broadcasted_iota(jnp.int32, (BLK, K), 1)
    wblk = jnp.zeros((BLK, K), jnp.float32)
    iblk = jnp.zeros((BLK, K), jnp.float32)
    for k in range(K):
        wblk = jnp.where(kiota == k, vals[k] / z, wblk)
        iblk = jnp.where(kiota == k, idxs[k], iblk)
    idx_ref[...] = iblk.astype(jnp.int32)
    wts_ref[...] = wblk

    neigh = _dot(wn.astype(jnp.bfloat16), basef_ref[...]) / z
    tok = _dotT(jnp.maximum(_dotT(neigh, c1w_ref[...]) + c1b_ref[...], 0.0),
                c2w_ref[...]) + c2b_ref[...]

    base = baseb_ref[...]
    cpl = _dotT(base, cphw_ref[...]) + cphb_ref[...]
    cpl = cpl - jnp.max(cpl, axis=1, keepdims=True)
    cpe = jnp.exp(cpl)
    cp = cpe / jnp.sum(cpe, axis=1, keepdims=True)

    gh = (_dotT(base, g1a_ref[...]) + _dotT(cp, g1b_ref[...])
          + _dotT(tok, g1c_ref[...]) + g1bias_ref[...])
    gh = jnp.maximum(gh, 0.0)
    gl = _dotT(gh, g2w_ref[...]) + g2b_ref[...]
    gl = gl - jnp.max(gl, axis=1, keepdims=True)
    ge = jnp.exp(gl)
    gp = ge / jnp.sum(ge, axis=1, keepdims=True)

    ent_ref[...] = -jnp.sum(gp * jnp.log(gp + 1e-8), axis=1, keepdims=True)

    cols = jax.lax.broadcasted_iota(jnp.int32, (BLK, 2 * HID), 1)
    factor = jnp.where(cols < HID, gp[:, 0:1], gp[:, 1:2])
    gated_ref[...] = base * factor


def _sc_gather_body(feat_hbm, idxf_hbm, wtsf_hbm, out_hbm,
                    idx0_v, idx1_v, wts_v, rows0_v, rows1_v, out_v,
                    sem0, sem1):
    wid = lax.axis_index("s") * 2 + lax.axis_index("c")
    row0 = wid * ROWS_W
    pltpu.sync_copy(wtsf_hbm.at[pl.ds(row0 * K, ROWS_W * K)], wts_v)

    idxb = [idx0_v, idx1_v]
    rowsb = [rows0_v, rows1_v]
    semb = [sem0, sem1]

    def issue(c, b):
        pltpu.sync_copy(idxf_hbm.at[pl.ds(row0 * K + c * CH * K, CH * K)],
                        idxb[b])
        pltpu.async_copy(feat_hbm.at[idxb[b]], rowsb[b], semb[b])

    issue(0, 0)

    def outer(c2, _):
        for b in range(2):
            c = c2 * 2 + b

            @pl.when(c + 1 < NCHUNK)
            def _():
                issue(c + 1, 1 - b)

            rows_v = rowsb[b]
            pltpu.make_async_copy(feat_hbm.at[idxb[b]], rows_v,
                                  semb[b]).wait()

            @plsc.parallel_loop(0, CH // 2, unroll=2)
            def rowpair(rp):
                wv = wts_v[pl.ds((c * CH + 2 * rp) * K, 2 * K)]
                for r01 in range(2):
                    ws = [wv[r01 * K + k] for k in range(K)]
                    r = (2 * rp + r01) * K
                    for h in range(D // 16):
                        hs = pl.ds(h * 16, 16)
                        p = [ws[2 * q] * rows_v[r + 2 * q, hs]
                             + ws[2 * q + 1] * rows_v[r + 2 * q + 1, hs]
                             for q in range(K // 2)]
                        out_v[2 * rp + r01, hs] = ((p[0] + p[1])
                                                   + (p[2] + p[3]))
            pltpu.sync_copy(out_v, out_hbm.at[pl.ds(row0 + c * CH, CH)])
        return 0

    lax.fori_loop(0, NCHUNK // 2, outer, 0)


def _sc_gather(feat, idx, wts):
    mesh = plsc.VectorSubcoreMesh(core_axis_name="c", subcore_axis_name="s")
    fn = functools.partial(
        pl.kernel,
        mesh=mesh,
        out_type=jax.ShapeDtypeStruct((B, D), jnp.float32),
        scratch_types=[
            pltpu.VMEM((CH * K,), jnp.int32),
            pltpu.VMEM((CH * K,), jnp.int32),
            pltpu.VMEM((ROWS_W * K,), jnp.float32),
            pltpu.VMEM((CH * K, D), jnp.float32),
            pltpu.VMEM((CH * K, D), jnp.float32),
            pltpu.VMEM((CH, D), jnp.float32),
            pltpu.SemaphoreType.DMA,
            pltpu.SemaphoreType.DMA,
        ],
    )(_sc_gather_body)
    return fn(feat, idx.reshape(B * K), wts.reshape(B * K))


def _head_body(upd_ref, gatedb_ref,
               gu1w_ref, gu1b_ref, gu2w_ref, gu2b_ref,
               cls1w_ref, cls1b_ref, bng_ref, bnb_ref,
               cls2w_ref, cls2b_ref, out_ref):
    upd = upd_ref[...]
    upd = _dotT(jnp.maximum(_dotT(upd, gu1w_ref[...]) + gu1b_ref[...], 0.0),
                gu2w_ref[...]) + gu2b_ref[...]
    fused = gatedb_ref[...] + ALPHA * upd
    h = _dotT(fused, cls1w_ref[...]) + cls1b_ref[...]
    h = (h / jnp.sqrt(1.0 + 1e-5)) * bng_ref[...] + bnb_ref[...]
    h = jnp.maximum(h, 0.0)
    out_ref[...] = _dotT(h, cls2w_ref[...]) + cls2b_ref[...]


def _full(shape):
    return pl.BlockSpec(shape, lambda i: (0, 0))


def _rows(w):
    return pl.BlockSpec((BLK, w), lambda i: (i, 0))


@jax.jit
def kernel(x, ln_v_w, ln_v_b, ln_t_w, ln_t_b, proj_v_w, proj_v_b, proj_t_w,
           proj_t_b, cph_w, cph_b, ctx1_w, ctx1_b, ctx2_w, ctx2_b, g1_w,
           g1_b, g2_w, g2_b, gu1_w, gu1_b, gu2_w, gu2_b, cls1_w, cls1_b,
           bn_g, bn_b, cls2_w, cls2_b):
    xc = x[:, 768:769]
    yc = x[:, 769:770]
    uc = x[:, 772:773]
    xr = xc.reshape(1, B)
    yr = yc.reshape(1, B)
    ur = uc.reshape(1, B)

    r1 = lambda v: v.reshape(1, -1)

    base, baseh = pl.pallas_call(
        _prep_body,
        grid=(NBLK,),
        in_specs=[_rows(773)] + [_full((1, 512))] * 2
                 + [_full((1, 256))] * 2
                 + [_full((HID, 512)), _full((1, HID)),
                    _full((HID, 256)), _full((1, HID))],
        out_specs=[_rows(2 * HID), _rows(2 * HID)],
        out_shape=[jax.ShapeDtypeStruct((B, 2 * HID), jnp.float32),
                   jax.ShapeDtypeStruct((B, 2 * HID), jnp.bfloat16)],
    )(x, r1(ln_v_w), r1(ln_v_b), r1(ln_t_w), r1(ln_t_b),
      proj_v_w, r1(proj_v_b), proj_t_w, r1(proj_t_b))

    g1a = g1_w[:, 0:2 * HID]
    g1b = g1_w[:, 2 * HID:2 * HID + NC]
    g1c = g1_w[:, 2 * HID + NC:]

    idx, wts, gated, ent = pl.pallas_call(
        _core_body,
        grid=(NBLK,),
        in_specs=[_rows(1)] * 3 + [_full((1, B))] * 3
                 + [_full((B, 2 * HID)), _rows(2 * HID),
                    _full((NC, 2 * HID)), _full((1, NC)),
                    _full((HID, 2 * HID)), _full((1, HID)),
                    _full((HID, HID)), _full((1, HID)),
                    _full((128, 2 * HID)), _full((128, NC)),
                    _full((128, HID)), _full((1, 128)),
                    _full((2, 128)), _full((1, 2))],
        out_specs=[_rows(K), _rows(K), _rows(2 * HID), _rows(1)],
        out_shape=[jax.ShapeDtypeStruct((B, K), jnp.int32),
                   jax.ShapeDtypeStruct((B, K), jnp.float32),
                   jax.ShapeDtypeStruct((B, 2 * HID), jnp.float32),
                   jax.ShapeDtypeStruct((B, 1), jnp.float32)],
    )(xc, yc, uc, xr, yr, ur, baseh, base, cph_w, r1(cph_b),
      ctx1_w, r1(ctx1_b), ctx2_w, r1(ctx2_b),
      g1a, g1b, g1c, r1(g1_b), g2_w, r1(g2_b))

    upd_raw = _sc_gather(gated, idx, wts)

    logits = pl.pallas_call(
        _head_body,
        grid=(NBLK,),
        in_specs=[_rows(2 * HID), _rows(2 * HID),
                  _full((2 * HID, 2 * HID)), _full((1, 2 * HID)),
                  _full((2 * HID, 2 * HID)), _full((1, 2 * HID)),
                  _full((HID, 2 * HID)), _full((1, HID)),
                  _full((1, HID)), _full((1, HID)),
                  _full((NC, HID)), _full((1, NC))],
        out_specs=_rows(NC),
        out_shape=jax.ShapeDtypeStruct((B, NC), jnp.float32),
    )(upd_raw, gated, gu1_w, r1(gu1_b), gu2_w, r1(gu2_b),
      cls1_w, r1(cls1_b), r1(bn_g), r1(bn_b), cls2_w, r1(cls2_b))

    ent_loss = jnp.mean(ent) * 0.01
    return logits, ent_loss


# speedup vs baseline: 1.4335x; 1.0542x over previous
import functools
import jax
import jax.numpy as jnp
from jax import lax
from jax.experimental import pallas as pl
from jax.experimental.pallas import tpu as pltpu
from jax.experimental.pallas import tpu_sc as plsc

B = 4096
HID = 128
NC = 5
K = 8
ALPHA = 0.5
BLK = 512
NBLK = B // BLK
BIGF = 1e30

NWORK = 32
ROWS_W = B // NWORK
CH = 16
NCHUNK = ROWS_W // CH
D = 2 * HID


def _dotT(a, b):
    return jax.lax.dot_general(a, b, (((1,), (1,)), ((), ())),
                               preferred_element_type=jnp.float32)


def _dot(a, b):
    return jax.lax.dot_general(a, b, (((1,), (0,)), ((), ())),
                               preferred_element_type=jnp.float32)


def _prep_body(x_ref, lvw_ref, lvb_ref, ltw_ref, ltb_ref,
               pvw_ref, pvb_ref, ptw_ref, ptb_ref, base_ref, baseh_ref):
    xv = x_ref[:, 0:512]
    xt = x_ref[:, 512:768]

    def ln(v, w, b):
        mu = jnp.mean(v, axis=1, keepdims=True)
        var = jnp.mean((v - mu) ** 2, axis=1, keepdims=True)
        return (v - mu) / jnp.sqrt(var + 1e-5) * w + b

    nv = ln(xv, lvw_ref[...], lvb_ref[...])
    nt = ln(xt, ltw_ref[...], ltb_ref[...])
    fv = _dotT(nv, pvw_ref[...]) + pvb_ref[...]
    ft = _dotT(nt, ptw_ref[...]) + ptb_ref[...]
    base = jnp.concatenate([fv, ft], axis=1)
    base_ref[...] = base
    baseh_ref[...] = base.astype(jnp.bfloat16)


def _core_body(xc_ref, yc_ref, uc_ref, xr_ref, yr_ref, ur_ref,
               basef_ref, baseb_ref,
               cphw_ref, cphb_ref, c1w_ref, c1b_ref, c2w_ref, c2b_ref,
               g1a_ref, g1b_ref, g1c_ref, g1bias_ref, g2w_ref, g2b_ref,
               idx_ref, wts_ref, gated_ref, ent_ref):
    pid = pl.program_id(0)
    dx = xc_ref[...] - xr_ref[...]
    dy = yc_ref[...] - yr_ref[...]
    d2 = dx * dx + dy * dy

    jota = jax.lax.broadcasted_iota(jnp.int32, (BLK, B), 1)
    row_id = jax.lax.broadcasted_iota(jnp.int32, (BLK, B), 0) + pid * BLK
    notself = jota != row_id
    eq = uc_ref[...] == ur_ref[...]
    same = jnp.logical_and(eq, notself)
    has_n = jnp.sum(same.astype(jnp.float32), axis=1, keepdims=True) > 0.0
    valid = jnp.logical_and(notself, jnp.logical_or(~has_n, same))
    d2m = jnp.where(valid, d2, BIGF)
    fiota = jota.astype(jnp.float32)

    v1 = None
    z = jnp.zeros((BLK, 1), jnp.float32)
    wn = jnp.zeros((BLK, B), jnp.float32)
    vals = []
    idxs = []
    for k in range(K):
        m = jnp.min(d2m, axis=1, keepdims=True)
        cand = jnp.where(d2m == m, fiota, BIGF)
        j = jnp.min(cand, axis=1, keepdims=True)
        onehot = fiota == j
        vk = -jnp.sqrt(jnp.maximum(m, 1e-12))
        if k == 0:
            v1 = vk
            ek = jnp.ones((BLK, 1), jnp.float32)
        else:
            ek = jnp.exp(vk - v1)
        z = z + ek
        wn = wn + jnp.where(onehot, ek, 0.0)
        vals.append(ek)
        idxs.append(j)
        if k < K - 1:
            d2m = jnp.where(onehot, BIGF, d2m)

    kiota = jax.lax.broadcasted_iota(jnp.int32, (BLK, K), 1)
    wblk = jnp.zeros((BLK, K), jnp.float32)
    iblk = jnp.zeros((BLK, K), jnp.float32)
    for k in range(K):
        wblk = jnp.where(kiota == k, vals[k] / z, wblk)
        iblk = jnp.where(kiota == k, idxs[k], iblk)
    idx_ref[...] = iblk.astype(jnp.int32)
    wts_ref[...] = wblk

    neigh = _dot(wn.astype(jnp.bfloat16), basef_ref[...]) / z
    tok = _dotT(jnp.maximum(_dotT(neigh, c1w_ref[...]) + c1b_ref[...], 0.0),
                c2w_ref[...]) + c2b_ref[...]

    base = baseb_ref[...]
    cpl = _dotT(base, cphw_ref[...]) + cphb_ref[...]
    cpl = cpl - jnp.max(cpl, axis=1, keepdims=True)
    cpe = jnp.exp(cpl)
    cp = cpe / jnp.sum(cpe, axis=1, keepdims=True)

    gh = (_dotT(base, g1a_ref[...]) + _dotT(cp, g1b_ref[...])
          + _dotT(tok, g1c_ref[...]) + g1bias_ref[...])
    gh = jnp.maximum(gh, 0.0)
    gl = _dotT(gh, g2w_ref[...]) + g2b_ref[...]
    gl = gl - jnp.max(gl, axis=1, keepdims=True)
    ge = jnp.exp(gl)
    gp = ge / jnp.sum(ge, axis=1, keepdims=True)

    ent_ref[...] = -jnp.sum(gp * jnp.log(gp + 1e-8), axis=1, keepdims=True)

    cols = jax.lax.broadcasted_iota(jnp.int32, (BLK, 2 * HID), 1)
    factor = jnp.where(cols < HID, gp[:, 0:1], gp[:, 1:2])
    gated_ref[...] = base * factor


def _sc_gather_body(feat_hbm, idxf_hbm, wtsf_hbm, out_hbm,
                    idx0_v, idx1_v, wts_v, rows0_v, rows1_v, out_v,
                    sem0, sem1):
    wid = lax.axis_index("s") * 2 + lax.axis_index("c")
    row0 = wid * ROWS_W
    pltpu.sync_copy(wtsf_hbm.at[pl.ds(row0 * K, ROWS_W * K)], wts_v)

    idxb = [idx0_v, idx1_v]
    rowsb = [rows0_v, rows1_v]
    semb = [sem0, sem1]

    def issue(c, b):
        pltpu.sync_copy(idxf_hbm.at[pl.ds(row0 * K + c * CH * K, CH * K)],
                        idxb[b])
        pltpu.async_copy(feat_hbm.at[idxb[b]], rowsb[b], semb[b])

    issue(0, 0)

    def outer(c2, _):
        for b in range(2):
            c = c2 * 2 + b

            @pl.when(c + 1 < NCHUNK)
            def _():
                issue(c + 1, 1 - b)

            rows_v = rowsb[b]
            pltpu.make_async_copy(feat_hbm.at[idxb[b]], rows_v,
                                  semb[b]).wait()

            @plsc.parallel_loop(0, CH // 2, unroll=2)
            def rowpair(rp):
                wv = wts_v[pl.ds((c * CH + 2 * rp) * K, 2 * K)]
                for r01 in range(2):
                    ws = [wv[r01 * K + k] for k in range(K)]
                    r = (2 * rp + r01) * K
                    for h in range(D // 16):
                        hs = pl.ds(h * 16, 16)
                        p = [ws[2 * q] * rows_v[r + 2 * q, hs]
                             + ws[2 * q + 1] * rows_v[r + 2 * q + 1, hs]
                             for q in range(K // 2)]
                        out_v[2 * rp + r01, hs] = ((p[0] + p[1])
                                                   + (p[2] + p[3]))
            pltpu.sync_copy(out_v, out_hbm.at[pl.ds(row0 + c * CH, CH)])
        return 0

    lax.fori_loop(0, NCHUNK // 2, outer, 0)


def _sc_gather(feat, idx, wts):
    mesh = plsc.VectorSubcoreMesh(core_axis_name="c", subcore_axis_name="s")
    fn = functools.partial(
        pl.kernel,
        mesh=mesh,
        out_type=jax.ShapeDtypeStruct((B, D), jnp.float32),
        scratch_types=[
            pltpu.VMEM((CH * K,), jnp.int32),
            pltpu.VMEM((CH * K,), jnp.int32),
            pltpu.VMEM((ROWS_W * K,), jnp.float32),
            pltpu.VMEM((CH * K, D), jnp.float32),
            pltpu.VMEM((CH * K, D), jnp.float32),
            pltpu.VMEM((CH, D), jnp.float32),
            pltpu.SemaphoreType.DMA,
            pltpu.SemaphoreType.DMA,
        ],
    )(_sc_gather_body)
    return fn(feat, idx.reshape(B * K), wts.reshape(B * K))


def _head_body(upd_ref, gatedb_ref,
               gu1w_ref, gu1b_ref, gu2w_ref, gu2b_ref,
               cls1w_ref, cls1b_ref, bng_ref, bnb_ref,
               cls2w_ref, cls2b_ref, out_ref):
    upd = upd_ref[...]
    upd = _dotT(jnp.maximum(_dotT(upd, gu1w_ref[...]) + gu1b_ref[...], 0.0),
                gu2w_ref[...]) + gu2b_ref[...]
    fused = gatedb_ref[...] + ALPHA * upd
    h = _dotT(fused, cls1w_ref[...]) + cls1b_ref[...]
    h = (h / jnp.sqrt(1.0 + 1e-5)) * bng_ref[...] + bnb_ref[...]
    h = jnp.maximum(h, 0.0)
    out_ref[...] = _dotT(h, cls2w_ref[...]) + cls2b_ref[...]


def _full(shape):
    return pl.BlockSpec(shape, lambda i: (0, 0))


def _rows(w):
    return pl.BlockSpec((BLK, w), lambda i: (i, 0))


@jax.jit
def kernel(x, ln_v_w, ln_v_b, ln_t_w, ln_t_b, proj_v_w, proj_v_b, proj_t_w,
           proj_t_b, cph_w, cph_b, ctx1_w, ctx1_b, ctx2_w, ctx2_b, g1_w,
           g1_b, g2_w, g2_b, gu1_w, gu1_b, gu2_w, gu2_b, cls1_w, cls1_b,
           bn_g, bn_b, cls2_w, cls2_b):
    xc = x[:, 768:769]
    yc = x[:, 769:770]
    uc = x[:, 772:773]
    xr = xc.reshape(1, B)
    yr = yc.reshape(1, B)
    ur = uc.reshape(1, B)

    r1 = lambda v: v.reshape(1, -1)

    base, baseh = pl.pallas_call(
        _prep_body,
        grid=(NBLK,),
        in_specs=[_rows(773)] + [_full((1, 512))] * 2
                 + [_full((1, 256))] * 2
                 + [_full((HID, 512)), _full((1, HID)),
                    _full((HID, 256)), _full((1, HID))],
        out_specs=[_rows(2 * HID), _rows(2 * HID)],
        out_shape=[jax.ShapeDtypeStruct((B, 2 * HID), jnp.float32),
                   jax.ShapeDtypeStruct((B, 2 * HID), jnp.bfloat16)],
    )(x, r1(ln_v_w), r1(ln_v_b), r1(ln_t_w), r1(ln_t_b),
      proj_v_w, r1(proj_v_b), proj_t_w, r1(proj_t_b))

    g1a = g1_w[:, 0:2 * HID]
    g1b = g1_w[:, 2 * HID:2 * HID + NC]
    g1c = g1_w[:, 2 * HID + NC:]

    idx, wts, gated, ent = pl.pallas_call(
        _core_body,
        grid=(NBLK,),
        in_specs=[_rows(1)] * 3 + [_full((1, B))] * 3
                 + [_full((B, 2 * HID)), _rows(2 * HID),
                    _full((NC, 2 * HID)), _full((1, NC)),
                    _full((HID, 2 * HID)), _full((1, HID)),
                    _full((HID, HID)), _full((1, HID)),
                    _full((128, 2 * HID)), _full((128, NC)),
                    _full((128, HID)), _full((1, 128)),
                    _full((2, 128)), _full((1, 2))],
        out_specs=[_rows(K), _rows(K), _rows(2 * HID), _rows(1)],
        out_shape=[jax.ShapeDtypeStruct((B, K), jnp.int32),
                   jax.ShapeDtypeStruct((B, K), jnp.float32),
                   jax.ShapeDtypeStruct((B, 2 * HID), jnp.float32),
                   jax.ShapeDtypeStruct((B, 1), jnp.float32)],
    )(xc, yc, uc, xr, yr, ur, baseh, base, cph_w, r1(cph_b),
      ctx1_w, r1(ctx1_b), ctx2_w, r1(ctx2_b),
      g1a, g1b, g1c, r1(g1_b), g2_w, r1(g2_b))

    upd_raw = _sc_gather(gated, idx, wts)

    logits = pl.pallas_call(
        _head_body,
        grid=(NBLK,),
        in_specs=[_rows(2 * HID), _rows(2 * HID),
                  _full((2 * HID, 2 * HID)), _full((1, 2 * HID)),
                  _full((2 * HID, 2 * HID)), _full((1, 2 * HID)),
                  _full((HID, 2 * HID)), _full((1, HID)),
                  _full((1, HID)), _full((1, HID)),
                  _full((NC, HID)), _full((1, NC))],
        out_specs=_rows(NC),
        out_shape=jax.ShapeDtypeStruct((B, NC), jnp.float32),
    )(upd_raw, gated, gu1_w, r1(gu1_b), gu2_w, r1(gu2_b),
      cls1_w, r1(cls1_b), r1(bn_g), r1(bn_b), cls2_w, r1(cls2_b))

    ent_loss = jnp.mean(ent) * 0.01
    return logits, ent_loss
